# baseline (speedup 1.0000x reference)
import functools
import math

import jax
import jax.numpy as jnp
from jax import lax
from jax.experimental import pallas as pl
from jax.experimental.pallas import tpu as pltpu
from jax.experimental.pallas import tpu_sc as plsc

N = 10000
E = 320000
DIN = 128
H = 512
DOUT = 128
EPS = 1e-5

NC = 2
NS = 16
NW = NC * NS
K = 64
EPT = E // NW
NBLK = 10
CBLK = 16
CPT = NBLK * CBLK
PADE = CPT * K - EPT
NTRI = (CBLK - 1) // 3
N_PAD = 10240
ROWS_PER_TILE = N_PAD // NS


def _lane_splat(vec, lane):
    idx = jnp.full((16, 1), lane, jnp.int32)
    return lax.gather(
        vec, idx,
        lax.GatherDimensionNumbers(offset_dims=(), collapsed_slice_dims=(0,),
                                   start_index_map=(0,)),
        (1,), mode=lax.GatherScatterMode.PROMISE_IN_BOUNDS)


def _sc_agg(src2d, dst2d, ew2d, table):
    D = table.shape[1]

    mesh = plsc.VectorSubcoreMesh(core_axis_name="c", subcore_axis_name="s")

    @functools.partial(
        pl.kernel,
        mesh=mesh,
        out_type=jax.ShapeDtypeStruct((NC, N_PAD, D), jnp.float32),
        scratch_types=[
            pltpu.VMEM((CBLK, K), jnp.int32),
            pltpu.VMEM((CBLK, K), jnp.int32),
            pltpu.VMEM((CBLK, K), jnp.float32),
            pltpu.VMEM((K, D), jnp.float32),
            pltpu.VMEM((K, D), jnp.float32),
            pltpu.VMEM((K, D), jnp.float32),
            pltpu.VMEM_SHARED((N_PAD, D), jnp.float32),
            pltpu.SemaphoreType.DMA,
            pltpu.SemaphoreType.DMA,
            pltpu.SemaphoreType.DMA,
            pltpu.SemaphoreType.DMA,
            pltpu.SemaphoreType.DMA,
            pltpu.SemaphoreType.DMA,
        ],
    )
    def agg_kernel(src_hbm, dst_hbm, ew_hbm, table_hbm, out_hbm,
                   src_v, dst_v, ew_v, rows0, rows1, rows2, acc,
                   g0, g1, g2, s0, s1, s2):
        cid = lax.axis_index("c")
        sid = lax.axis_index("s")
        wid = sid * NC + cid

        nzseg = D // 16

        def zrow(r, carry):
            for c in range(nzseg):
                rows0[r, pl.ds(c * 16, 16)] = jnp.zeros((16,), jnp.float32)
            return carry

        lax.fori_loop(0, K, zrow, 0)
        for r8 in range(ROWS_PER_TILE // K):
            pltpu.sync_copy(
                rows0, acc.at[pl.ds(sid * ROWS_PER_TILE + r8 * K, K)])

        plsc.subcore_barrier()

        nseg = D // 16
        ngrp = K // 16

        def scale_rows(rows_v, j):
            def group_body(g, carry2):
                wv = ew_v[j, pl.ds(g * 16, 16)]
                for e16 in range(16):
                    w = _lane_splat(wv, e16)
                    e = g * 16 + e16
                    for c in range(nseg):
                        sl = pl.ds(c * 16, 16)
                        rows_v[e, sl] = rows_v[e, sl] * w
                return carry2

            lax.fori_loop(0, ngrp, group_body, 0)

        def start_gather(j, rows_v, sem):
            pltpu.async_copy(table_hbm.at[src_v.at[j]], rows_v, sem)

        def wait_gather(j, rows_v, sem):
            pltpu.make_async_copy(table_hbm.at[src_v.at[j]], rows_v, sem).wait()

        def start_scatter(j, rows_v, sem):
            pltpu.async_copy(rows_v, acc.at[dst_v.at[j]], sem, add=True)

        def wait_scatter(j, rows_v, sem):
            pltpu.make_async_copy(rows_v, acc.at[dst_v.at[j]], sem).wait()

        bufs = [(rows0, g0, s0), (rows1, g1, s1), (rows2, g2, s2)]

        def block_body(b, carry0):
            pltpu.sync_copy(src_hbm.at[wid, b], src_v)
            pltpu.sync_copy(dst_hbm.at[wid, b], dst_v)
            pltpu.sync_copy(ew_hbm.at[wid, b], ew_v)

            start_gather(0, rows0, g0)
            start_gather(1, rows1, g1)
            wait_gather(0, rows0, g0)
            scale_rows(rows0, 0)
            start_scatter(0, rows0, s0)
            start_gather(2, rows2, g2)

            def tri_body(t, carry):
                for u in range(3):
                    j = 3 * t + 1 + u
                    buf, gs, ss = bufs[(1 + u) % 3]
                    pbuf, pgs, pss = bufs[u % 3]
                    wait_gather(j, buf, gs)
                    scale_rows(buf, j)
                    start_scatter(j, buf, ss)
                    wait_scatter(j - 1, pbuf, pss)

                    @pl.when(j + 2 <= CBLK - 1)
                    def _():
                        start_gather(j + 2, pbuf, pgs)
                return carry

            lax.fori_loop(0, NTRI, tri_body, 0)

            wait_scatter(CBLK - 1, rows0, s0)
            return carry0

        lax.fori_loop(0, NBLK, block_body, 0)

        plsc.subcore_barrier()

        pltpu.sync_copy(acc.at[pl.ds(sid * ROWS_PER_TILE, ROWS_PER_TILE)],
                        out_hbm.at[cid, pl.ds(sid * ROWS_PER_TILE, ROWS_PER_TILE)])

    return agg_kernel(src2d, dst2d, ew2d, table)


def _gelu_exact(x):
    return 0.5 * x * (1.0 + lax.erf(x * (1.0 / math.sqrt(2.0))))


def _tc1_body(aggp_ref, W1_ref, b1_ref, g1_ref, be1_ref, Wa1_ref, ba1_ref,
              Wa2_ref, ba2_ref, W2_ref, p_ref):
    agg = aggp_ref[0] + aggp_ref[1]
    t = jnp.dot(agg, W1_ref[...], preferred_element_type=jnp.float32) + b1_ref[...]
    h = _gelu_exact(t)
    m = jnp.mean(h, axis=0, keepdims=True)
    v = jnp.mean((h - m) * (h - m), axis=0, keepdims=True)
    h = (h - m) * lax.rsqrt(v + EPS) * g1_ref[...] + be1_ref[...]
    a = jnp.dot(h, Wa1_ref[...], preferred_element_type=jnp.float32) + ba1_ref[...]
    a = jnp.maximum(a, 0.0)
    att = jax.nn.sigmoid(
        jnp.dot(a, Wa2_ref[...], preferred_element_type=jnp.float32) + ba2_ref[...])
    h = h * att
    p_ref[...] = jnp.dot(h, W2_ref[...], preferred_element_type=jnp.float32)


def _tc1(aggp, W1, b1, g1, be1, Wa1, ba1, Wa2, ba2, W2):
    return pl.pallas_call(
        _tc1_body,
        grid=(1,),
        in_specs=[
            pl.BlockSpec((NC, N, DIN), lambda i: (0, 0, 0)),
            pl.BlockSpec((DIN, H), lambda i: (0, 0)),
            pl.BlockSpec((1, H), lambda i: (0, 0)),
            pl.BlockSpec((1, H), lambda i: (0, 0)),
            pl.BlockSpec((1, H), lambda i: (0, 0)),
            pl.BlockSpec((H, H // 4), lambda i: (0, 0)),
            pl.BlockSpec((1, H // 4), lambda i: (0, 0)),
            pl.BlockSpec((H // 4, H), lambda i: (0, 0)),
            pl.BlockSpec((1, H), lambda i: (0, 0)),
            pl.BlockSpec((H, DOUT), lambda i: (0, 0)),
        ],
        out_specs=pl.BlockSpec((N, DOUT), lambda i: (0, 0)),
        out_shape=jax.ShapeDtypeStruct((N, DOUT), jnp.float32),
    )(aggp, W1, b1, g1, be1, Wa1, ba1, Wa2, ba2, W2)


def _tc3_body(aggp_ref, b2_ref, g2_ref, be2_ref, out_ref):
    y = _gelu_exact(aggp_ref[0] + aggp_ref[1] + b2_ref[...])
    m = jnp.mean(y, axis=0, keepdims=True)
    v = jnp.mean((y - m) * (y - m), axis=0, keepdims=True)
    out_ref[...] = (y - m) * lax.rsqrt(v + EPS) * g2_ref[...] + be2_ref[...]


def _tc3(aggp, b2, g2, be2):
    return pl.pallas_call(
        _tc3_body,
        grid=(1,),
        in_specs=[
            pl.BlockSpec((NC, N, DOUT), lambda i: (0, 0, 0)),
            pl.BlockSpec((1, DOUT), lambda i: (0, 0)),
            pl.BlockSpec((1, DOUT), lambda i: (0, 0)),
            pl.BlockSpec((1, DOUT), lambda i: (0, 0)),
        ],
        out_specs=pl.BlockSpec((N, DOUT), lambda i: (0, 0)),
        out_shape=jax.ShapeDtypeStruct((N, DOUT), jnp.float32),
    )(aggp, b2, g2, be2)


def kernel(x, edge_index, edge_weight, W1, b1, g1, be1, Wa1, ba1, Wa2, ba2,
           W2, b2, g2, be2):
    pad = ((0, 0), (0, PADE))
    src2d = jnp.pad(edge_index[0].reshape(NW, EPT), pad).reshape(NW, NBLK, CBLK, K)
    dst2d = jnp.pad(edge_index[1].reshape(NW, EPT), pad).reshape(NW, NBLK, CBLK, K)
    ew2d = jnp.pad(edge_weight.reshape(NW, EPT), pad).reshape(NW, NBLK, CBLK, K)

    aggp1 = _sc_agg(src2d, dst2d, ew2d, x)
    p = _tc1(aggp1, W1, b1.reshape(1, H), g1.reshape(1, H), be1.reshape(1, H),
             Wa1, ba1.reshape(1, H // 4), Wa2, ba2.reshape(1, H), W2)
    aggp2 = _sc_agg(src2d, dst2d, ew2d, p)
    out = _tc3(aggp2, b2.reshape(1, DOUT), g2.reshape(1, DOUT),
               be2.reshape(1, DOUT))
    return out


# speedup vs baseline: 2.3707x; 2.3707x over previous
import functools
import math

import jax
import jax.numpy as jnp
from jax import lax
from jax.experimental import pallas as pl
from jax.experimental.pallas import tpu as pltpu
from jax.experimental.pallas import tpu_sc as plsc

N = 10000
E = 320000
DIN = 128
H = 512
DOUT = 128
EPS = 1e-5

NC = 2
NS = 16
NW = NC * NS
K = 80
EPT = E // NW
NBLK = 5
CBLK = 25
CPT = NBLK * CBLK
N_PAD = 10240
ROWS_PER_TILE = N_PAD // NS


def _lane_splat(vec, lane):
    idx = jnp.full((16, 1), lane, jnp.int32)
    return lax.gather(
        vec, idx,
        lax.GatherDimensionNumbers(offset_dims=(), collapsed_slice_dims=(0,),
                                   start_index_map=(0,)),
        (1,), mode=lax.GatherScatterMode.PROMISE_IN_BOUNDS)


def _sc_agg(src2d, dst2d, ew2d, table):
    D = table.shape[1]

    mesh = plsc.VectorSubcoreMesh(core_axis_name="c", subcore_axis_name="s")

    @functools.partial(
        pl.kernel,
        mesh=mesh,
        out_type=jax.ShapeDtypeStruct((NC, N_PAD, D), jnp.float32),
        scratch_types=[
            pltpu.VMEM((CBLK, K), jnp.int32),
            pltpu.VMEM((CBLK, K), jnp.int32),
            pltpu.VMEM((CBLK, K), jnp.float32),
            pltpu.VMEM((K, D), jnp.float32),
            pltpu.VMEM((K, D), jnp.float32),
            pltpu.VMEM_SHARED((N_PAD, D), jnp.float32),
            pltpu.SemaphoreType.DMA,
            pltpu.SemaphoreType.DMA,
            pltpu.SemaphoreType.DMA,
            pltpu.SemaphoreType.DMA,
        ],
    )
    def agg_kernel(src_hbm, dst_hbm, ew_hbm, table_hbm, out_hbm,
                   src_v, dst_v, ew_v, rows0, rows1, acc, sem0, sem1, ssem0, ssem1):
        cid = lax.axis_index("c")
        sid = lax.axis_index("s")
        wid = sid * NC + cid

        nzseg = D // 16

        def zrow(r, carry):
            for c in range(nzseg):
                rows0[r, pl.ds(c * 16, 16)] = jnp.zeros((16,), jnp.float32)
            return carry

        lax.fori_loop(0, K, zrow, 0)
        for r8 in range(ROWS_PER_TILE // K):
            pltpu.sync_copy(
                rows0, acc.at[pl.ds(sid * ROWS_PER_TILE + r8 * K, K)])

        plsc.subcore_barrier()

        nseg = D // 16
        ngrp = K // 16

        def scale_rows(rows_v, j):
            def group_body(g, carry2):
                wv = ew_v[j, pl.ds(g * 16, 16)]
                for e16 in range(16):
                    w = _lane_splat(wv, e16)
                    e = g * 16 + e16
                    for c in range(nseg):
                        sl = pl.ds(c * 16, 16)
                        rows_v[e, sl] = rows_v[e, sl] * w
                return carry2

            lax.fori_loop(0, ngrp, group_body, 0)

        def start_gather(j, rows_v, sem):
            pltpu.async_copy(table_hbm.at[src_v.at[j]], rows_v, sem)

        def wait_gather(j, rows_v, sem):
            pltpu.make_async_copy(table_hbm.at[src_v.at[j]], rows_v, sem).wait()

        def start_scatter(j, rows_v, sem):
            pltpu.async_copy(rows_v, acc.at[dst_v.at[j]], sem, add=True)

        def wait_scatter(j, rows_v, sem):
            pltpu.make_async_copy(rows_v, acc.at[dst_v.at[j]], sem).wait()

        NPAIR = (CBLK - 1) // 2

        def block_body(b, carry0):
            pltpu.sync_copy(src_hbm.at[wid, b], src_v)
            pltpu.sync_copy(dst_hbm.at[wid, b], dst_v)
            pltpu.sync_copy(ew_hbm.at[wid, b], ew_v)

            start_gather(0, rows0, sem0)

            def pair_body(i, carry):
                j0 = 2 * i
                wait_gather(j0, rows0, sem0)

                @pl.when(i > 0)
                def _():
                    wait_scatter(j0 - 1, rows1, ssem1)

                start_gather(j0 + 1, rows1, sem1)
                scale_rows(rows0, j0)
                start_scatter(j0, rows0, ssem0)
                wait_gather(j0 + 1, rows1, sem1)
                wait_scatter(j0, rows0, ssem0)
                start_gather(j0 + 2, rows0, sem0)
                scale_rows(rows1, j0 + 1)
                start_scatter(j0 + 1, rows1, ssem1)
                return carry

            lax.fori_loop(0, NPAIR, pair_body, 0)

            wait_gather(CBLK - 1, rows0, sem0)
            wait_scatter(CBLK - 2, rows1, ssem1)
            scale_rows(rows0, CBLK - 1)
            start_scatter(CBLK - 1, rows0, ssem0)
            wait_scatter(CBLK - 1, rows0, ssem0)
            return carry0

        lax.fori_loop(0, NBLK, block_body, 0)

        plsc.subcore_barrier()

        pltpu.sync_copy(acc.at[pl.ds(sid * ROWS_PER_TILE, ROWS_PER_TILE)],
                        out_hbm.at[cid, pl.ds(sid * ROWS_PER_TILE, ROWS_PER_TILE)])

    return agg_kernel(src2d, dst2d, ew2d, table)


def _gelu_exact(x):
    return 0.5 * x * (1.0 + lax.erf(x * (1.0 / math.sqrt(2.0))))


def _tc1_body(aggp_ref, W1_ref, b1_ref, g1_ref, be1_ref, Wa1_ref, ba1_ref,
              Wa2_ref, ba2_ref, W2_ref, p_ref):
    agg = aggp_ref[0] + aggp_ref[1]
    t = jnp.dot(agg, W1_ref[...], preferred_element_type=jnp.float32) + b1_ref[...]
    h = _gelu_exact(t)
    m = jnp.mean(h, axis=0, keepdims=True)
    v = jnp.mean((h - m) * (h - m), axis=0, keepdims=True)
    h = (h - m) * lax.rsqrt(v + EPS) * g1_ref[...] + be1_ref[...]
    a = jnp.dot(h, Wa1_ref[...], preferred_element_type=jnp.float32) + ba1_ref[...]
    a = jnp.maximum(a, 0.0)
    att = jax.nn.sigmoid(
        jnp.dot(a, Wa2_ref[...], preferred_element_type=jnp.float32) + ba2_ref[...])
    h = h * att
    p_ref[...] = jnp.dot(h, W2_ref[...], preferred_element_type=jnp.float32)


def _tc1(aggp, W1, b1, g1, be1, Wa1, ba1, Wa2, ba2, W2):
    return pl.pallas_call(
        _tc1_body,
        grid=(1,),
        in_specs=[
            pl.BlockSpec((NC, N, DIN), lambda i: (0, 0, 0)),
            pl.BlockSpec((DIN, H), lambda i: (0, 0)),
            pl.BlockSpec((1, H), lambda i: (0, 0)),
            pl.BlockSpec((1, H), lambda i: (0, 0)),
            pl.BlockSpec((1, H), lambda i: (0, 0)),
            pl.BlockSpec((H, H // 4), lambda i: (0, 0)),
            pl.BlockSpec((1, H // 4), lambda i: (0, 0)),
            pl.BlockSpec((H // 4, H), lambda i: (0, 0)),
            pl.BlockSpec((1, H), lambda i: (0, 0)),
            pl.BlockSpec((H, DOUT), lambda i: (0, 0)),
        ],
        out_specs=pl.BlockSpec((N, DOUT), lambda i: (0, 0)),
        out_shape=jax.ShapeDtypeStruct((N, DOUT), jnp.float32),
    )(aggp, W1, b1, g1, be1, Wa1, ba1, Wa2, ba2, W2)


def _tc3_body(aggp_ref, b2_ref, g2_ref, be2_ref, out_ref):
    y = _gelu_exact(aggp_ref[0] + aggp_ref[1] + b2_ref[...])
    m = jnp.mean(y, axis=0, keepdims=True)
    v = jnp.mean((y - m) * (y - m), axis=0, keepdims=True)
    out_ref[...] = (y - m) * lax.rsqrt(v + EPS) * g2_ref[...] + be2_ref[...]


def _tc3(aggp, b2, g2, be2):
    return pl.pallas_call(
        _tc3_body,
        grid=(1,),
        in_specs=[
            pl.BlockSpec((NC, N, DOUT), lambda i: (0, 0, 0)),
            pl.BlockSpec((1, DOUT), lambda i: (0, 0)),
            pl.BlockSpec((1, DOUT), lambda i: (0, 0)),
            pl.BlockSpec((1, DOUT), lambda i: (0, 0)),
        ],
        out_specs=pl.BlockSpec((N, DOUT), lambda i: (0, 0)),
        out_shape=jax.ShapeDtypeStruct((N, DOUT), jnp.float32),
    )(aggp, b2, g2, be2)


def kernel(x, edge_index, edge_weight, W1, b1, g1, be1, Wa1, ba1, Wa2, ba2,
           W2, b2, g2, be2):
    src2d = edge_index[0].reshape(NW, NBLK, CBLK, K)
    dst2d = edge_index[1].reshape(NW, NBLK, CBLK, K)
    ew2d = edge_weight.reshape(NW, NBLK, CBLK, K)

    aggp1 = _sc_agg(src2d, dst2d, ew2d, x)
    p = _tc1(aggp1, W1, b1.reshape(1, H), g1.reshape(1, H), be1.reshape(1, H),
             Wa1, ba1.reshape(1, H // 4), Wa2, ba2.reshape(1, H), W2)
    aggp2 = _sc_agg(src2d, dst2d, ew2d, p)
    out = _tc3(aggp2, b2.reshape(1, DOUT), g2.reshape(1, DOUT),
               be2.reshape(1, DOUT))
    return out


# speedup vs baseline: 2.3773x; 1.0028x over previous
import functools
import math

import jax
import jax.numpy as jnp
from jax import lax
from jax.experimental import pallas as pl
from jax.experimental.pallas import tpu as pltpu
from jax.experimental.pallas import tpu_sc as plsc

N = 10000
E = 320000
DIN = 128
H = 512
DOUT = 128
EPS = 1e-5

NC = 2
NS = 16
NW = NC * NS
K = 80
EPT = E // NW
NBLK = 5
CBLK = 25
CPT = NBLK * CBLK
N_PAD = 10240
ROWS_PER_TILE = N_PAD // NS


def _lane_splat(vec, lane):
    idx = jnp.full((16, 1), lane, jnp.int32)
    return lax.gather(
        vec, idx,
        lax.GatherDimensionNumbers(offset_dims=(), collapsed_slice_dims=(0,),
                                   start_index_map=(0,)),
        (1,), mode=lax.GatherScatterMode.PROMISE_IN_BOUNDS)


def _sc_agg(src2d, dst2d, ew2d, table):
    D = table.shape[1]

    mesh = plsc.VectorSubcoreMesh(core_axis_name="c", subcore_axis_name="s")

    @functools.partial(
        pl.kernel,
        mesh=mesh,
        out_type=jax.ShapeDtypeStruct((NC, N_PAD, D), jnp.float32),
        scratch_types=[
            pltpu.VMEM((CBLK, K), jnp.int32),
            pltpu.VMEM((CBLK, K), jnp.int32),
            pltpu.VMEM((CBLK, K), jnp.float32),
            pltpu.VMEM((K, D), jnp.float32),
            pltpu.VMEM((K, D), jnp.float32),
            pltpu.VMEM_SHARED((N_PAD, D), jnp.float32),
            pltpu.SemaphoreType.DMA,
            pltpu.SemaphoreType.DMA,
        ],
    )
    def agg_kernel(src_hbm, dst_hbm, ew_hbm, table_hbm, out_hbm,
                   src_v, dst_v, ew_v, rows0, rows1, acc, sem0, sem1):
        cid = lax.axis_index("c")
        sid = lax.axis_index("s")
        wid = sid * NC + cid

        nzseg = D // 16

        def zrow(r, carry):
            for c in range(nzseg):
                rows0[r, pl.ds(c * 16, 16)] = jnp.zeros((16,), jnp.float32)
            return carry

        lax.fori_loop(0, K, zrow, 0)
        for r8 in range(ROWS_PER_TILE // K):
            pltpu.sync_copy(
                rows0, acc.at[pl.ds(sid * ROWS_PER_TILE + r8 * K, K)])

        plsc.subcore_barrier()

        nseg = D // 16
        ngrp = K // 16

        def scale_rows(rows_v, j):
            def group_body(g, carry2):
                wv = ew_v[j, pl.ds(g * 16, 16)]
                for e16 in range(16):
                    w = _lane_splat(wv, e16)
                    e = g * 16 + e16
                    for c in range(nseg):
                        sl = pl.ds(c * 16, 16)
                        rows_v[e, sl] = rows_v[e, sl] * w
                return carry2

            lax.fori_loop(0, ngrp, group_body, 0)

        def start_gather(j, rows_v, sem):
            pltpu.async_copy(table_hbm.at[src_v.at[j]], rows_v, sem)

        def wait_gather(j, rows_v, sem):
            pltpu.make_async_copy(table_hbm.at[src_v.at[j]], rows_v, sem).wait()

        def scatter_add(j, rows_v):
            pltpu.sync_copy(rows_v, acc.at[dst_v.at[j]], add=True)

        NPAIR = (CBLK - 1) // 2

        def block_body(b, carry0):
            pltpu.sync_copy(src_hbm.at[wid, b], src_v)
            pltpu.sync_copy(dst_hbm.at[wid, b], dst_v)
            pltpu.sync_copy(ew_hbm.at[wid, b], ew_v)

            start_gather(0, rows0, sem0)

            def pair_body(i, carry):
                j0 = 2 * i
                wait_gather(j0, rows0, sem0)
                start_gather(j0 + 1, rows1, sem1)
                scale_rows(rows0, j0)
                scatter_add(j0, rows0)
                wait_gather(j0 + 1, rows1, sem1)
                start_gather(j0 + 2, rows0, sem0)
                scale_rows(rows1, j0 + 1)
                scatter_add(j0 + 1, rows1)
                return carry

            lax.fori_loop(0, NPAIR, pair_body, 0)

            wait_gather(CBLK - 1, rows0, sem0)
            scale_rows(rows0, CBLK - 1)
            scatter_add(CBLK - 1, rows0)
            return carry0

        lax.fori_loop(0, NBLK, block_body, 0)

        plsc.subcore_barrier()

        pltpu.sync_copy(acc.at[pl.ds(sid * ROWS_PER_TILE, ROWS_PER_TILE)],
                        out_hbm.at[cid, pl.ds(sid * ROWS_PER_TILE, ROWS_PER_TILE)])

    return agg_kernel(src2d, dst2d, ew2d, table)


def _gelu_exact(x):
    return 0.5 * x * (1.0 + lax.erf(x * (1.0 / math.sqrt(2.0))))


def _tc1_body(aggp_ref, W1_ref, b1_ref, g1_ref, be1_ref, Wa1_ref, ba1_ref,
              Wa2_ref, ba2_ref, W2_ref, p_ref):
    agg = aggp_ref[0] + aggp_ref[1]
    t = jnp.dot(agg, W1_ref[...], preferred_element_type=jnp.float32) + b1_ref[...]
    h = _gelu_exact(t)
    m = jnp.mean(h, axis=0, keepdims=True)
    v = jnp.mean((h - m) * (h - m), axis=0, keepdims=True)
    h = (h - m) * lax.rsqrt(v + EPS) * g1_ref[...] + be1_ref[...]
    a = jnp.dot(h, Wa1_ref[...], preferred_element_type=jnp.float32) + ba1_ref[...]
    a = jnp.maximum(a, 0.0)
    att = jax.nn.sigmoid(
        jnp.dot(a, Wa2_ref[...], preferred_element_type=jnp.float32) + ba2_ref[...])
    h = h * att
    p_ref[...] = jnp.dot(h, W2_ref[...], preferred_element_type=jnp.float32)


def _tc1(aggp, W1, b1, g1, be1, Wa1, ba1, Wa2, ba2, W2):
    return pl.pallas_call(
        _tc1_body,
        grid=(1,),
        in_specs=[
            pl.BlockSpec((NC, N, DIN), lambda i: (0, 0, 0)),
            pl.BlockSpec((DIN, H), lambda i: (0, 0)),
            pl.BlockSpec((1, H), lambda i: (0, 0)),
            pl.BlockSpec((1, H), lambda i: (0, 0)),
            pl.BlockSpec((1, H), lambda i: (0, 0)),
            pl.BlockSpec((H, H // 4), lambda i: (0, 0)),
            pl.BlockSpec((1, H // 4), lambda i: (0, 0)),
            pl.BlockSpec((H // 4, H), lambda i: (0, 0)),
            pl.BlockSpec((1, H), lambda i: (0, 0)),
            pl.BlockSpec((H, DOUT), lambda i: (0, 0)),
        ],
        out_specs=pl.BlockSpec((N, DOUT), lambda i: (0, 0)),
        out_shape=jax.ShapeDtypeStruct((N, DOUT), jnp.float32),
    )(aggp, W1, b1, g1, be1, Wa1, ba1, Wa2, ba2, W2)


def _tc3_body(aggp_ref, b2_ref, g2_ref, be2_ref, out_ref):
    y = _gelu_exact(aggp_ref[0] + aggp_ref[1] + b2_ref[...])
    m = jnp.mean(y, axis=0, keepdims=True)
    v = jnp.mean((y - m) * (y - m), axis=0, keepdims=True)
    out_ref[...] = (y - m) * lax.rsqrt(v + EPS) * g2_ref[...] + be2_ref[...]


def _tc3(aggp, b2, g2, be2):
    return pl.pallas_call(
        _tc3_body,
        grid=(1,),
        in_specs=[
            pl.BlockSpec((NC, N, DOUT), lambda i: (0, 0, 0)),
            pl.BlockSpec((1, DOUT), lambda i: (0, 0)),
            pl.BlockSpec((1, DOUT), lambda i: (0, 0)),
            pl.BlockSpec((1, DOUT), lambda i: (0, 0)),
        ],
        out_specs=pl.BlockSpec((N, DOUT), lambda i: (0, 0)),
        out_shape=jax.ShapeDtypeStruct((N, DOUT), jnp.float32),
    )(aggp, b2, g2, be2)


def kernel(x, edge_index, edge_weight, W1, b1, g1, be1, Wa1, ba1, Wa2, ba2,
           W2, b2, g2, be2):
    src2d = edge_index[0].reshape(NW, NBLK, CBLK, K)
    dst2d = edge_index[1].reshape(NW, NBLK, CBLK, K)
    ew2d = edge_weight.reshape(NW, NBLK, CBLK, K)

    aggp1 = _sc_agg(src2d, dst2d, ew2d, x)
    p = _tc1(aggp1, W1, b1.reshape(1, H), g1.reshape(1, H), be1.reshape(1, H),
             Wa1, ba1.reshape(1, H // 4), Wa2, ba2.reshape(1, H), W2)
    aggp2 = _sc_agg(src2d, dst2d, ew2d, p)
    out = _tc3(aggp2, b2.reshape(1, DOUT), g2.reshape(1, DOUT),
               be2.reshape(1, DOUT))
    return out
